# baseline (speedup 1.0000x reference)
import functools

import jax
import jax.numpy as jnp
from jax import lax
from jax.experimental import pallas as pl
from jax.experimental.pallas import tpu as pltpu
from jax.experimental.pallas import tpu_sc as plsc

NC = 2
NS = 16
NW = NC * NS

_mesh = lambda: plsc.VectorSubcoreMesh(core_axis_name="c", subcore_axis_name="s")


def _sc_gather(nodes_pk, senders, receivers):
    n, d = nodes_pk.shape
    e = senders.shape[0]
    per_w = e // NW
    ch = 200
    n_it = per_w // ch

    @functools.partial(
        pl.kernel,
        mesh=_mesh(),
        out_type=jax.ShapeDtypeStruct((e, 2 * d), jnp.int32),
        scratch_types=[
            pltpu.VMEM((ch,), jnp.int32),
            pltpu.VMEM((ch,), jnp.int32),
            pltpu.VMEM((ch, d), jnp.int32),
            pltpu.VMEM((ch, d), jnp.int32),
            pltpu.SemaphoreType.DMA,
            pltpu.SemaphoreType.DMA,
        ],
    )
    def k(nodes_hbm, s_hbm, r_hbm, g_hbm, sidx, ridx, sbuf, rbuf, sem_s, sem_r):
        wid = lax.axis_index("s") * NC + lax.axis_index("c")
        base = wid * per_w

        def body(i, carry):
            e0 = base + i * ch
            pltpu.sync_copy(s_hbm.at[pl.ds(e0, ch)], sidx)
            pltpu.sync_copy(r_hbm.at[pl.ds(e0, ch)], ridx)
            cs = pltpu.async_copy(nodes_hbm.at[sidx], sbuf, sem_s)
            cr = pltpu.async_copy(nodes_hbm.at[ridx], rbuf, sem_r)
            cs.wait()
            cr.wait()
            pltpu.sync_copy(sbuf, g_hbm.at[pl.ds(e0, ch), pl.ds(0, d)])
            pltpu.sync_copy(rbuf, g_hbm.at[pl.ds(e0, ch), pl.ds(d, d)])
            return carry

        lax.fori_loop(0, n_it, body, 0)

    return k(nodes_pk, senders, receivers)


def _sc_segsum(vals, idx, n_seg):
    f = vals.shape[1]
    e = idx.shape[0]
    fb = f // NC
    per_t = e // NS
    ch = 200
    n_it = per_t // ch
    n_pad = ((n_seg + NS * 8 - 1) // (NS * 8)) * (NS * 8)
    rows_t = n_pad // NS
    zeros = jnp.zeros((rows_t, fb), jnp.float32)

    @functools.partial(
        pl.kernel,
        mesh=_mesh(),
        out_type=jax.ShapeDtypeStruct((n_pad, f), jnp.float32),
        scratch_types=[
            pltpu.VMEM((ch,), jnp.int32),
            pltpu.VMEM((ch, fb), jnp.float32),
            pltpu.VMEM_SHARED((n_pad, fb), jnp.float32),
        ],
    )
    def k(v_hbm, i_hbm, z_hbm, out_hbm, idxbuf, rowsbuf, table):
        c = lax.axis_index("c")
        sid = lax.axis_index("s")
        r0 = sid * rows_t
        pltpu.sync_copy(z_hbm, table.at[pl.ds(r0, rows_t)])
        plsc.subcore_barrier()

        base = sid * per_t

        def body(i, carry):
            e0 = base + i * ch
            pltpu.sync_copy(i_hbm.at[pl.ds(e0, ch)], idxbuf)
            pltpu.sync_copy(v_hbm.at[pl.ds(e0, ch), pl.ds(c * fb, fb)], rowsbuf)
            pltpu.sync_copy(rowsbuf, table.at[idxbuf], add=True)
            return carry

        lax.fori_loop(0, n_it, body, 0)
        plsc.subcore_barrier()
        pltpu.sync_copy(table.at[pl.ds(r0, rows_t)],
                        out_hbm.at[pl.ds(r0, rows_t), pl.ds(c * fb, fb)])

    return k(vals, idx, zeros)[:n_seg]


def _edge_body(e_ref, g_ref, w1e_ref, w1sr_ref, w1g_ref, gl_ref, b1_ref,
               w2_ref, b2_ref, o_ref):
    bf = jnp.bfloat16
    acc = jnp.dot(e_ref[...].astype(bf), w1e_ref[...].astype(bf),
                  preferred_element_type=jnp.float32)
    acc += jnp.dot(g_ref[...].astype(bf), w1sr_ref[...].astype(bf),
                   preferred_element_type=jnp.float32)
    acc += jnp.dot(gl_ref[...], w1g_ref[...], preferred_element_type=jnp.float32)
    h = jnp.maximum(acc + b1_ref[...], 0.0)
    o_ref[...] = (jnp.dot(h.astype(bf), w2_ref[...].astype(bf),
                          preferred_element_type=jnp.float32)
                  + b2_ref[...])


def _tc_edge(edges, g, w1e, w1sr, w1g, gl, b1, w2, b2):
    e, de = edges.shape
    dg = g.shape[1]
    h = w1e.shape[1]
    eo = w2.shape[1]
    blk = 640
    grid = e // blk
    full = lambda i: (0, 0)
    return pl.pallas_call(
        _edge_body,
        grid=(grid,),
        in_specs=[
            pl.BlockSpec((blk, de), lambda i: (i, 0)),
            pl.BlockSpec((blk, dg), lambda i: (i, 0)),
            pl.BlockSpec(w1e.shape, full),
            pl.BlockSpec(w1sr.shape, full),
            pl.BlockSpec(w1g.shape, full),
            pl.BlockSpec(gl.shape, full),
            pl.BlockSpec(b1.shape, full),
            pl.BlockSpec(w2.shape, full),
            pl.BlockSpec(b2.shape, full),
        ],
        out_specs=pl.BlockSpec((blk, eo), lambda i: (i, 0)),
        out_shape=jax.ShapeDtypeStruct((e, eo), jnp.float32),
    )(edges, g, w1e, w1sr, w1g, gl, b1, w2, b2)


def _node_body(n_ref, s_ref, r_ref, wa_ref, wb_ref, wc_ref, wg_ref, gl_ref,
               b1_ref, w2_ref, b2_ref, o_ref):
    bf = jnp.bfloat16
    acc = jnp.dot(n_ref[...].astype(bf), wa_ref[...].astype(bf),
                  preferred_element_type=jnp.float32)
    acc += jnp.dot(s_ref[...].astype(bf), wb_ref[...].astype(bf),
                   preferred_element_type=jnp.float32)
    acc += jnp.dot(r_ref[...].astype(bf), wc_ref[...].astype(bf),
                   preferred_element_type=jnp.float32)
    acc += jnp.dot(gl_ref[...], wg_ref[...], preferred_element_type=jnp.float32)
    h = jnp.maximum(acc + b1_ref[...], 0.0)
    o_ref[...] = (jnp.dot(h.astype(bf), w2_ref[...].astype(bf),
                          preferred_element_type=jnp.float32)
                  + b2_ref[...] + n_ref[...])


def _tc_node(nodes, agg_s, agg_r, wa, wb, wc, wg, gl, b1, w2, b2):
    n, dn = nodes.shape
    no = w2.shape[1]
    blk = 1000
    grid = n // blk
    full = lambda i: (0, 0)
    return pl.pallas_call(
        _node_body,
        grid=(grid,),
        in_specs=[
            pl.BlockSpec((blk, dn), lambda i: (i, 0)),
            pl.BlockSpec((blk, agg_s.shape[1]), lambda i: (i, 0)),
            pl.BlockSpec((blk, agg_r.shape[1]), lambda i: (i, 0)),
            pl.BlockSpec(wa.shape, full),
            pl.BlockSpec(wb.shape, full),
            pl.BlockSpec(wc.shape, full),
            pl.BlockSpec(wg.shape, full),
            pl.BlockSpec(gl.shape, full),
            pl.BlockSpec(b1.shape, full),
            pl.BlockSpec(w2.shape, full),
            pl.BlockSpec(b2.shape, full),
        ],
        out_specs=pl.BlockSpec((blk, no), lambda i: (i, 0)),
        out_shape=jax.ShapeDtypeStruct((n, no), jnp.float32),
    )(nodes, agg_s, agg_r, wa, wb, wc, wg, gl, b1, w2, b2)


def kernel(nodes, edges, globals_, senders, receivers,
           W_e1, b_e1, W_e2, b_e2, W_n1, b_n1, W_n2, b_n2):
    n, dn = nodes.shape
    e = senders.shape[0]
    de = edges.shape[1]
    dg = globals_.shape[1]
    gl = globals_.reshape(1, dg).astype(jnp.float32)

    w1e = W_e1[:de]
    w1sr = W_e1[de:de + 2 * dn]
    w1g = W_e1[de + 2 * dn:]

    nodes_pk = lax.bitcast_convert_type(
        nodes.astype(jnp.bfloat16).reshape(n, dn // 2, 2), jnp.int32)
    g_pk = _sc_gather(nodes_pk, senders, receivers)
    g = lax.bitcast_convert_type(g_pk, jnp.bfloat16).reshape(e, 2 * dn)
    new_edges = _tc_edge(edges, g, w1e, w1sr, w1g, gl,
                         b_e1.reshape(1, -1), W_e2, b_e2.reshape(1, -1))

    agg_s = _sc_segsum(new_edges, senders, n)
    agg_r = _sc_segsum(new_edges, receivers, n)

    eo = new_edges.shape[1]
    wa = W_n1[:dn]
    wb = W_n1[dn:dn + eo]
    wc = W_n1[dn + eo:dn + 2 * eo]
    wg = W_n1[dn + 2 * eo:]

    out_nodes = _tc_node(nodes, agg_s, agg_r, wa, wb, wc, wg, gl,
                         b_n1.reshape(1, -1), W_n2, b_n2.reshape(1, -1))
    return (out_nodes, edges, globals_)


# speedup vs baseline: 2.2933x; 2.2933x over previous
import functools

import jax
import jax.numpy as jnp
from jax import lax
from jax.experimental import pallas as pl
from jax.experimental.pallas import tpu as pltpu
from jax.experimental.pallas import tpu_sc as plsc

NC = 2
NS = 16
NW = NC * NS

_mesh = lambda: plsc.VectorSubcoreMesh(core_axis_name="c", subcore_axis_name="s")


def _sc_gather(nodes, senders, receivers):
    n, d = nodes.shape
    e = senders.shape[0]
    per_w = e // NW
    ch = 40
    ns = 5
    n_grp = per_w // (ch * ns)

    @functools.partial(
        pl.kernel,
        mesh=_mesh(),
        out_type=jax.ShapeDtypeStruct((e, 2 * d), jnp.float32),
        scratch_types=(
            [pltpu.VMEM((per_w,), jnp.int32)] * 2
            + [pltpu.VMEM((ch, d), jnp.float32)] * (2 * ns)
            + [pltpu.SemaphoreType.DMA] * (2 * ns)
        ),
    )
    def k(nodes_hbm, s_hbm, r_hbm, g_hbm, sidx, ridx, *bufs_sems):
        sbufs = bufs_sems[0:ns]
        rbufs = bufs_sems[ns:2 * ns]
        gsems = bufs_sems[2 * ns:3 * ns]
        wsems = bufs_sems[3 * ns:4 * ns]
        wid = lax.axis_index("s") * NC + lax.axis_index("c")
        base = wid * per_w
        pltpu.sync_copy(s_hbm.at[pl.ds(base, per_w)], sidx)
        pltpu.sync_copy(r_hbm.at[pl.ds(base, per_w)], ridx)

        def group(g, _):
            g0 = g * (ch * ns)
            gh = []
            for s in range(ns):
                o = g0 + s * ch
                gh.append((
                    pltpu.async_copy(
                        nodes_hbm.at[sidx.at[pl.ds(o, ch)]], sbufs[s],
                        gsems[s]),
                    pltpu.async_copy(
                        nodes_hbm.at[ridx.at[pl.ds(o, ch)]], rbufs[s],
                        gsems[s]),
                ))
            wh = []
            for s in range(ns):
                o = base + g0 + s * ch
                gh[s][0].wait()
                gh[s][1].wait()
                wh.append((
                    pltpu.async_copy(
                        sbufs[s], g_hbm.at[pl.ds(o, ch), pl.ds(0, d)],
                        wsems[s]),
                    pltpu.async_copy(
                        rbufs[s], g_hbm.at[pl.ds(o, ch), pl.ds(d, d)],
                        wsems[s]),
                ))
            for s in range(ns):
                wh[s][0].wait()
                wh[s][1].wait()
            return _

        lax.fori_loop(0, n_grp, group, 0)

    return k(nodes, senders, receivers)


def _sc_segsum(vals, idx, n_seg):
    f = vals.shape[1]
    e = idx.shape[0]
    fb = f // NC
    per_t = e // NS
    chs = (104, 96)
    pair = sum(chs)
    n_grp = per_t // pair
    n_pad = ((n_seg + NS * 8 - 1) // (NS * 8)) * (NS * 8)
    rows_t = n_pad // NS
    zeros = jnp.zeros((rows_t, fb), jnp.float32)

    @functools.partial(
        pl.kernel,
        mesh=_mesh(),
        out_type=jax.ShapeDtypeStruct((n_pad, f), jnp.float32),
        scratch_types=[
            pltpu.VMEM((chs[0],), jnp.int32),
            pltpu.VMEM((chs[1],), jnp.int32),
            pltpu.VMEM((chs[0], fb), jnp.float32),
            pltpu.VMEM((chs[1], fb), jnp.float32),
            pltpu.VMEM_SHARED((n_pad, fb), jnp.float32),
            pltpu.SemaphoreType.DMA,
            pltpu.SemaphoreType.DMA,
        ],
    )
    def k(v_hbm, i_hbm, z_hbm, out_hbm, ib0, ib1, rb0, rb1, table,
          vsem0, vsem1):
        c = lax.axis_index("c")
        sid = lax.axis_index("s")
        r0 = sid * rows_t
        pltpu.sync_copy(z_hbm, table.at[pl.ds(r0, rows_t)])
        plsc.subcore_barrier()

        ibufs = (ib0, ib1)
        rbufs = (rb0, rb1)
        vsems = (vsem0, vsem1)
        base = sid * per_t

        def group(g, _):
            g0 = base + g * pair
            vh = []
            for s in range(2):
                e0 = g0 + s * chs[0]
                pltpu.sync_copy(i_hbm.at[pl.ds(e0, chs[s])], ibufs[s])
                vh.append(pltpu.async_copy(
                    v_hbm.at[pl.ds(e0, chs[s]), pl.ds(c * fb, fb)], rbufs[s],
                    vsems[s]))
            for s in range(2):
                vh[s].wait()
                pltpu.sync_copy(rbufs[s], table.at[ibufs[s]], add=True)
            return _

        lax.fori_loop(0, n_grp, group, 0)

        plsc.subcore_barrier()
        pltpu.sync_copy(table.at[pl.ds(r0, rows_t)],
                        out_hbm.at[pl.ds(r0, rows_t), pl.ds(c * fb, fb)])

    return k(vals, idx, zeros)[:n_seg]


def _edge_body(e_ref, g_ref, w1e_ref, w1sr_ref, w1g_ref, gl_ref, b1_ref,
               w2_ref, b2_ref, o_ref):
    acc = jnp.dot(e_ref[...], w1e_ref[...], preferred_element_type=jnp.float32)
    acc += jnp.dot(g_ref[...], w1sr_ref[...], preferred_element_type=jnp.float32)
    acc += jnp.dot(gl_ref[...], w1g_ref[...], preferred_element_type=jnp.float32)
    h = jnp.maximum(acc + b1_ref[...], 0.0)
    o_ref[...] = (jnp.dot(h, w2_ref[...], preferred_element_type=jnp.float32)
                  + b2_ref[...])


def _tc_edge(edges, g, w1e, w1sr, w1g, gl, b1, w2, b2):
    e, de = edges.shape
    dg = g.shape[1]
    eo = w2.shape[1]
    blk = 640
    grid = e // blk
    full = lambda i: (0, 0)
    return pl.pallas_call(
        _edge_body,
        grid=(grid,),
        in_specs=[
            pl.BlockSpec((blk, de), lambda i: (i, 0)),
            pl.BlockSpec((blk, dg), lambda i: (i, 0)),
            pl.BlockSpec(w1e.shape, full),
            pl.BlockSpec(w1sr.shape, full),
            pl.BlockSpec(w1g.shape, full),
            pl.BlockSpec(gl.shape, full),
            pl.BlockSpec(b1.shape, full),
            pl.BlockSpec(w2.shape, full),
            pl.BlockSpec(b2.shape, full),
        ],
        out_specs=pl.BlockSpec((blk, eo), lambda i: (i, 0)),
        out_shape=jax.ShapeDtypeStruct((e, eo), jnp.float32),
    )(edges, g, w1e, w1sr, w1g, gl, b1, w2, b2)


def _node_body(n_ref, s_ref, r_ref, wa_ref, wb_ref, wc_ref, wg_ref, gl_ref,
               b1_ref, w2_ref, b2_ref, o_ref):
    acc = jnp.dot(n_ref[...], wa_ref[...], preferred_element_type=jnp.float32)
    acc += jnp.dot(s_ref[...], wb_ref[...], preferred_element_type=jnp.float32)
    acc += jnp.dot(r_ref[...], wc_ref[...], preferred_element_type=jnp.float32)
    acc += jnp.dot(gl_ref[...], wg_ref[...], preferred_element_type=jnp.float32)
    h = jnp.maximum(acc + b1_ref[...], 0.0)
    o_ref[...] = (jnp.dot(h, w2_ref[...], preferred_element_type=jnp.float32)
                  + b2_ref[...] + n_ref[...])


def _tc_node(nodes, agg_s, agg_r, wa, wb, wc, wg, gl, b1, w2, b2):
    n, dn = nodes.shape
    no = w2.shape[1]
    blk = 1000
    grid = n // blk
    full = lambda i: (0, 0)
    return pl.pallas_call(
        _node_body,
        grid=(grid,),
        in_specs=[
            pl.BlockSpec((blk, dn), lambda i: (i, 0)),
            pl.BlockSpec((blk, agg_s.shape[1]), lambda i: (i, 0)),
            pl.BlockSpec((blk, agg_r.shape[1]), lambda i: (i, 0)),
            pl.BlockSpec(wa.shape, full),
            pl.BlockSpec(wb.shape, full),
            pl.BlockSpec(wc.shape, full),
            pl.BlockSpec(wg.shape, full),
            pl.BlockSpec(gl.shape, full),
            pl.BlockSpec(b1.shape, full),
            pl.BlockSpec(w2.shape, full),
            pl.BlockSpec(b2.shape, full),
        ],
        out_specs=pl.BlockSpec((blk, no), lambda i: (i, 0)),
        out_shape=jax.ShapeDtypeStruct((n, no), jnp.float32),
    )(nodes, agg_s, agg_r, wa, wb, wc, wg, gl, b1, w2, b2)


def kernel(nodes, edges, globals_, senders, receivers,
           W_e1, b_e1, W_e2, b_e2, W_n1, b_n1, W_n2, b_n2):
    n, dn = nodes.shape
    de = edges.shape[1]
    dg = globals_.shape[1]
    gl = globals_.reshape(1, dg).astype(jnp.float32)

    w1e = W_e1[:de]
    w1sr = W_e1[de:de + 2 * dn]
    w1g = W_e1[de + 2 * dn:]

    g = _sc_gather(nodes, senders, receivers)
    new_edges = _tc_edge(edges, g, w1e, w1sr, w1g, gl,
                         b_e1.reshape(1, -1), W_e2, b_e2.reshape(1, -1))

    agg_s = _sc_segsum(new_edges, senders, n)
    agg_r = _sc_segsum(new_edges, receivers, n)

    eo = new_edges.shape[1]
    wa = W_n1[:dn]
    wb = W_n1[dn:dn + eo]
    wc = W_n1[dn + eo:dn + 2 * eo]
    wg = W_n1[dn + 2 * eo:]

    out_nodes = _tc_node(nodes, agg_s, agg_r, wa, wb, wc, wg, gl,
                         b_n1.reshape(1, -1), W_n2, b_n2.reshape(1, -1))
    return (out_nodes, edges, globals_)


# speedup vs baseline: 2.3885x; 1.0415x over previous
import functools

import jax
import jax.numpy as jnp
from jax import lax
from jax.experimental import pallas as pl
from jax.experimental.pallas import tpu as pltpu
from jax.experimental.pallas import tpu_sc as plsc

NC = 2
NS = 16
NW = NC * NS

_mesh = lambda: plsc.VectorSubcoreMesh(core_axis_name="c", subcore_axis_name="s")


def _sc_gather(nodes, senders, receivers):
    n, d = nodes.shape
    dt = nodes.dtype
    e = senders.shape[0]
    per_w = e // NW
    ch = 40
    ns = 5
    n_grp = per_w // (ch * ns)

    @functools.partial(
        pl.kernel,
        mesh=_mesh(),
        out_type=jax.ShapeDtypeStruct((e, 2 * d), dt),
        scratch_types=(
            [pltpu.VMEM((per_w,), jnp.int32)] * 2
            + [pltpu.VMEM((ch, d), dt)] * (2 * ns)
            + [pltpu.SemaphoreType.DMA] * (2 * ns)
        ),
    )
    def k(nodes_hbm, s_hbm, r_hbm, g_hbm, sidx, ridx, *bufs_sems):
        sbufs = bufs_sems[0:ns]
        rbufs = bufs_sems[ns:2 * ns]
        gsems = bufs_sems[2 * ns:3 * ns]
        wsems = bufs_sems[3 * ns:4 * ns]
        wid = lax.axis_index("s") * NC + lax.axis_index("c")
        base = wid * per_w
        pltpu.sync_copy(s_hbm.at[pl.ds(base, per_w)], sidx)
        pltpu.sync_copy(r_hbm.at[pl.ds(base, per_w)], ridx)

        def group(g, _):
            g0 = g * (ch * ns)
            gh = []
            for s in range(ns):
                o = g0 + s * ch
                gh.append((
                    pltpu.async_copy(
                        nodes_hbm.at[sidx.at[pl.ds(o, ch)]], sbufs[s],
                        gsems[s]),
                    pltpu.async_copy(
                        nodes_hbm.at[ridx.at[pl.ds(o, ch)]], rbufs[s],
                        gsems[s]),
                ))
            wh = []
            for s in range(ns):
                o = base + g0 + s * ch
                gh[s][0].wait()
                gh[s][1].wait()
                wh.append((
                    pltpu.async_copy(
                        sbufs[s], g_hbm.at[pl.ds(o, ch), pl.ds(0, d)],
                        wsems[s]),
                    pltpu.async_copy(
                        rbufs[s], g_hbm.at[pl.ds(o, ch), pl.ds(d, d)],
                        wsems[s]),
                ))
            for s in range(ns):
                wh[s][0].wait()
                wh[s][1].wait()
            return _

        lax.fori_loop(0, n_grp, group, 0)

    return k(nodes, senders, receivers)


def _sc_segsum(vals, idx, n_seg):
    f = vals.shape[1]
    e = idx.shape[0]
    fb = f // NC
    per_t = e // NS
    chs = (104, 96)
    pair = sum(chs)
    n_grp = per_t // pair
    n_pad = ((n_seg + NS * 8 - 1) // (NS * 8)) * (NS * 8)
    rows_t = n_pad // NS
    zeros = jnp.zeros((rows_t, fb), jnp.float32)

    @functools.partial(
        pl.kernel,
        mesh=_mesh(),
        out_type=jax.ShapeDtypeStruct((n_pad, f), jnp.float32),
        scratch_types=[
            pltpu.VMEM((chs[0],), jnp.int32),
            pltpu.VMEM((chs[1],), jnp.int32),
            pltpu.VMEM((chs[0], fb), jnp.float32),
            pltpu.VMEM((chs[1], fb), jnp.float32),
            pltpu.VMEM_SHARED((n_pad, fb), jnp.float32),
            pltpu.SemaphoreType.DMA,
            pltpu.SemaphoreType.DMA,
        ],
    )
    def k(v_hbm, i_hbm, z_hbm, out_hbm, ib0, ib1, rb0, rb1, table,
          vsem0, vsem1):
        c = lax.axis_index("c")
        sid = lax.axis_index("s")
        r0 = sid * rows_t
        pltpu.sync_copy(z_hbm, table.at[pl.ds(r0, rows_t)])
        plsc.subcore_barrier()

        ibufs = (ib0, ib1)
        rbufs = (rb0, rb1)
        vsems = (vsem0, vsem1)
        base = sid * per_t

        def group(g, _):
            g0 = base + g * pair
            vh = []
            for s in range(2):
                e0 = g0 + s * chs[0]
                pltpu.sync_copy(i_hbm.at[pl.ds(e0, chs[s])], ibufs[s])
                vh.append(pltpu.async_copy(
                    v_hbm.at[pl.ds(e0, chs[s]), pl.ds(c * fb, fb)], rbufs[s],
                    vsems[s]))
            for s in range(2):
                vh[s].wait()
                pltpu.sync_copy(rbufs[s], table.at[ibufs[s]], add=True)
            return _

        lax.fori_loop(0, n_grp, group, 0)

        plsc.subcore_barrier()
        pltpu.sync_copy(table.at[pl.ds(r0, rows_t)],
                        out_hbm.at[pl.ds(r0, rows_t), pl.ds(c * fb, fb)])

    return k(vals, idx, zeros)[:n_seg]


def _edge_body(e_ref, gpk_ref, w1e_ref, we_ref, wo_ref, w1g_ref, gl_ref,
               b1_ref, w2_ref, b2_ref, o_ref):
    w = gpk_ref[...]
    lo = lax.bitcast_convert_type(w << 16, jnp.float32)
    hi = lax.bitcast_convert_type(w & jnp.int32(-65536), jnp.float32)
    acc = jnp.dot(e_ref[...], w1e_ref[...], preferred_element_type=jnp.float32)
    acc += jnp.dot(lo, we_ref[...], preferred_element_type=jnp.float32)
    acc += jnp.dot(hi, wo_ref[...], preferred_element_type=jnp.float32)
    acc += jnp.dot(gl_ref[...], w1g_ref[...], preferred_element_type=jnp.float32)
    h = jnp.maximum(acc + b1_ref[...], 0.0)
    o_ref[...] = (jnp.dot(h, w2_ref[...], preferred_element_type=jnp.float32)
                  + b2_ref[...])


def _tc_edge(edges, g_pk, w1e, we, wo, w1g, gl, b1, w2, b2):
    e, de = edges.shape
    dg = g_pk.shape[1]
    eo = w2.shape[1]
    blk = 640
    grid = e // blk
    full = lambda i: (0, 0)
    return pl.pallas_call(
        _edge_body,
        grid=(grid,),
        in_specs=[
            pl.BlockSpec((blk, de), lambda i: (i, 0)),
            pl.BlockSpec((blk, dg), lambda i: (i, 0)),
            pl.BlockSpec(w1e.shape, full),
            pl.BlockSpec(we.shape, full),
            pl.BlockSpec(wo.shape, full),
            pl.BlockSpec(w1g.shape, full),
            pl.BlockSpec(gl.shape, full),
            pl.BlockSpec(b1.shape, full),
            pl.BlockSpec(w2.shape, full),
            pl.BlockSpec(b2.shape, full),
        ],
        out_specs=pl.BlockSpec((blk, eo), lambda i: (i, 0)),
        out_shape=jax.ShapeDtypeStruct((e, eo), jnp.float32),
    )(edges, g_pk, w1e, we, wo, w1g, gl, b1, w2, b2)


def _node_body(n_ref, s_ref, r_ref, wa_ref, wb_ref, wc_ref, wg_ref, gl_ref,
               b1_ref, w2_ref, b2_ref, o_ref):
    acc = jnp.dot(n_ref[...], wa_ref[...], preferred_element_type=jnp.float32)
    acc += jnp.dot(s_ref[...], wb_ref[...], preferred_element_type=jnp.float32)
    acc += jnp.dot(r_ref[...], wc_ref[...], preferred_element_type=jnp.float32)
    acc += jnp.dot(gl_ref[...], wg_ref[...], preferred_element_type=jnp.float32)
    h = jnp.maximum(acc + b1_ref[...], 0.0)
    o_ref[...] = (jnp.dot(h, w2_ref[...], preferred_element_type=jnp.float32)
                  + b2_ref[...] + n_ref[...])


def _tc_node(nodes, agg_s, agg_r, wa, wb, wc, wg, gl, b1, w2, b2):
    n, dn = nodes.shape
    no = w2.shape[1]
    blk = 1000
    grid = n // blk
    full = lambda i: (0, 0)
    return pl.pallas_call(
        _node_body,
        grid=(grid,),
        in_specs=[
            pl.BlockSpec((blk, dn), lambda i: (i, 0)),
            pl.BlockSpec((blk, agg_s.shape[1]), lambda i: (i, 0)),
            pl.BlockSpec((blk, agg_r.shape[1]), lambda i: (i, 0)),
            pl.BlockSpec(wa.shape, full),
            pl.BlockSpec(wb.shape, full),
            pl.BlockSpec(wc.shape, full),
            pl.BlockSpec(wg.shape, full),
            pl.BlockSpec(gl.shape, full),
            pl.BlockSpec(b1.shape, full),
            pl.BlockSpec(w2.shape, full),
            pl.BlockSpec(b2.shape, full),
        ],
        out_specs=pl.BlockSpec((blk, no), lambda i: (i, 0)),
        out_shape=jax.ShapeDtypeStruct((n, no), jnp.float32),
    )(nodes, agg_s, agg_r, wa, wb, wc, wg, gl, b1, w2, b2)


def kernel(nodes, edges, globals_, senders, receivers,
           W_e1, b_e1, W_e2, b_e2, W_n1, b_n1, W_n2, b_n2):
    n, dn = nodes.shape
    de = edges.shape[1]
    dg = globals_.shape[1]
    gl = globals_.reshape(1, dg).astype(jnp.float32)

    w1e = W_e1[:de]
    w1sr = W_e1[de:de + 2 * dn]
    we = w1sr[0::2]
    wo = w1sr[1::2]
    w1g = W_e1[de + 2 * dn:]

    nodes_pk = lax.bitcast_convert_type(
        nodes.astype(jnp.bfloat16).reshape(n, dn // 2, 2), jnp.int32)
    g_pk = _sc_gather(nodes_pk, senders, receivers)
    new_edges = _tc_edge(edges, g_pk, w1e, we, wo, w1g, gl,
                         b_e1.reshape(1, -1), W_e2, b_e2.reshape(1, -1))

    agg_s = _sc_segsum(new_edges, senders, n)
    agg_r = _sc_segsum(new_edges, receivers, n)

    eo = new_edges.shape[1]
    wa = W_n1[:dn]
    wb = W_n1[dn:dn + eo]
    wc = W_n1[dn + eo:dn + 2 * eo]
    wg = W_n1[dn + 2 * eo:]

    out_nodes = _tc_node(nodes, agg_s, agg_r, wa, wb, wc, wg, gl,
                         b_n1.reshape(1, -1), W_n2, b_n2.reshape(1, -1))
    return (out_nodes, edges, globals_)


# speedup vs baseline: 2.4022x; 1.0057x over previous
import functools

import jax
import jax.numpy as jnp
from jax import lax
from jax.experimental import pallas as pl
from jax.experimental.pallas import tpu as pltpu
from jax.experimental.pallas import tpu_sc as plsc

NC = 2
NS = 16
NW = NC * NS

_mesh = lambda: plsc.VectorSubcoreMesh(core_axis_name="c", subcore_axis_name="s")


def _sc_gather(nodes, senders, receivers):
    n, d = nodes.shape
    dt = nodes.dtype
    e = senders.shape[0]
    per_w = e // NW
    ch = 40
    ns = 5
    n_grp = per_w // (ch * ns)

    @functools.partial(
        pl.kernel,
        mesh=_mesh(),
        out_type=jax.ShapeDtypeStruct((e, 2 * d), dt),
        scratch_types=(
            [pltpu.VMEM((per_w,), jnp.int32)] * 2
            + [pltpu.VMEM((ch, d), dt)] * (2 * ns)
            + [pltpu.SemaphoreType.DMA] * (2 * ns)
        ),
    )
    def k(nodes_hbm, s_hbm, r_hbm, g_hbm, sidx, ridx, *bufs_sems):
        sbufs = bufs_sems[0:ns]
        rbufs = bufs_sems[ns:2 * ns]
        gsems = bufs_sems[2 * ns:3 * ns]
        wsems = bufs_sems[3 * ns:4 * ns]
        wid = lax.axis_index("s") * NC + lax.axis_index("c")
        base = wid * per_w
        pltpu.sync_copy(s_hbm.at[pl.ds(base, per_w)], sidx)
        pltpu.sync_copy(r_hbm.at[pl.ds(base, per_w)], ridx)

        def group(g, _):
            g0 = g * (ch * ns)
            gh = []
            for s in range(ns):
                o = g0 + s * ch
                gh.append((
                    pltpu.async_copy(
                        nodes_hbm.at[sidx.at[pl.ds(o, ch)]], sbufs[s],
                        gsems[s]),
                    pltpu.async_copy(
                        nodes_hbm.at[ridx.at[pl.ds(o, ch)]], rbufs[s],
                        gsems[s]),
                ))
            wh = []
            for s in range(ns):
                o = base + g0 + s * ch
                gh[s][0].wait()
                gh[s][1].wait()
                wh.append((
                    pltpu.async_copy(
                        sbufs[s], g_hbm.at[pl.ds(o, ch), pl.ds(0, d)],
                        wsems[s]),
                    pltpu.async_copy(
                        rbufs[s], g_hbm.at[pl.ds(o, ch), pl.ds(d, d)],
                        wsems[s]),
                ))
            for s in range(ns):
                wh[s][0].wait()
                wh[s][1].wait()
            return _

        lax.fori_loop(0, n_grp, group, 0)

    return k(nodes, senders, receivers)


def _sc_segsum2(vals, idx_a, idx_b, n_seg):
    f = vals.shape[1]
    e = idx_a.shape[0]
    fb = f // NC
    per_t = e // NS
    chs = (104, 96)
    pair = sum(chs)
    n_grp = per_t // pair
    n_pad = ((n_seg + NS * 8 - 1) // (NS * 8)) * (NS * 8)
    rows_t = n_pad // NS
    zeros = jnp.zeros((rows_t, fb), jnp.float32)
    out_sd = jax.ShapeDtypeStruct((n_pad, f), jnp.float32)

    @functools.partial(
        pl.kernel,
        mesh=_mesh(),
        out_type=(out_sd, out_sd),
        scratch_types=[
            pltpu.VMEM((chs[0],), jnp.int32),
            pltpu.VMEM((chs[1],), jnp.int32),
            pltpu.VMEM((chs[0], fb), jnp.float32),
            pltpu.VMEM((chs[1], fb), jnp.float32),
            pltpu.VMEM_SHARED((n_pad, fb), jnp.float32),
            pltpu.SemaphoreType.DMA,
            pltpu.SemaphoreType.DMA,
        ],
    )
    def k(v_hbm, ia_hbm, ib_hbm, z_hbm, outa_hbm, outb_hbm,
          ib0, ib1, rb0, rb1, table, vsem0, vsem1):
        c = lax.axis_index("c")
        sid = lax.axis_index("s")
        r0 = sid * rows_t
        ibufs = (ib0, ib1)
        rbufs = (rb0, rb1)
        vsems = (vsem0, vsem1)
        base = sid * per_t

        for i_hbm, out_hbm in ((ia_hbm, outa_hbm), (ib_hbm, outb_hbm)):
            pltpu.sync_copy(z_hbm, table.at[pl.ds(r0, rows_t)])
            plsc.subcore_barrier()

            def group(g, _):
                g0 = base + g * pair
                vh = []
                for s in range(2):
                    e0 = g0 + s * chs[0]
                    pltpu.sync_copy(i_hbm.at[pl.ds(e0, chs[s])], ibufs[s])
                    vh.append(pltpu.async_copy(
                        v_hbm.at[pl.ds(e0, chs[s]), pl.ds(c * fb, fb)],
                        rbufs[s], vsems[s]))
                for s in range(2):
                    vh[s].wait()
                    pltpu.sync_copy(rbufs[s], table.at[ibufs[s]], add=True)
                return _

            lax.fori_loop(0, n_grp, group, 0)

            plsc.subcore_barrier()
            pltpu.sync_copy(table.at[pl.ds(r0, rows_t)],
                            out_hbm.at[pl.ds(r0, rows_t), pl.ds(c * fb, fb)])

    agg_a, agg_b = k(vals, idx_a, idx_b, zeros)
    return agg_a[:n_seg], agg_b[:n_seg]


def _edge_body(e_ref, gpk_ref, w1e_ref, we_ref, wo_ref, w1g_ref, gl_ref,
               b1_ref, w2_ref, b2_ref, o_ref):
    w = gpk_ref[...]
    lo = lax.bitcast_convert_type(w << 16, jnp.float32)
    hi = lax.bitcast_convert_type(w & jnp.int32(-65536), jnp.float32)
    acc = jnp.dot(e_ref[...], w1e_ref[...], preferred_element_type=jnp.float32)
    acc += jnp.dot(lo, we_ref[...], preferred_element_type=jnp.float32)
    acc += jnp.dot(hi, wo_ref[...], preferred_element_type=jnp.float32)
    acc += jnp.dot(gl_ref[...], w1g_ref[...], preferred_element_type=jnp.float32)
    h = jnp.maximum(acc + b1_ref[...], 0.0)
    o_ref[...] = (jnp.dot(h, w2_ref[...], preferred_element_type=jnp.float32)
                  + b2_ref[...])


def _tc_edge(edges, g_pk, w1e, we, wo, w1g, gl, b1, w2, b2):
    e, de = edges.shape
    dg = g_pk.shape[1]
    eo = w2.shape[1]
    blk = 640
    grid = e // blk
    full = lambda i: (0, 0)
    return pl.pallas_call(
        _edge_body,
        grid=(grid,),
        in_specs=[
            pl.BlockSpec((blk, de), lambda i: (i, 0)),
            pl.BlockSpec((blk, dg), lambda i: (i, 0)),
            pl.BlockSpec(w1e.shape, full),
            pl.BlockSpec(we.shape, full),
            pl.BlockSpec(wo.shape, full),
            pl.BlockSpec(w1g.shape, full),
            pl.BlockSpec(gl.shape, full),
            pl.BlockSpec(b1.shape, full),
            pl.BlockSpec(w2.shape, full),
            pl.BlockSpec(b2.shape, full),
        ],
        out_specs=pl.BlockSpec((blk, eo), lambda i: (i, 0)),
        out_shape=jax.ShapeDtypeStruct((e, eo), jnp.float32),
    )(edges, g_pk, w1e, we, wo, w1g, gl, b1, w2, b2)


def _node_body(n_ref, s_ref, r_ref, wa_ref, wb_ref, wc_ref, wg_ref, gl_ref,
               b1_ref, w2_ref, b2_ref, o_ref):
    acc = jnp.dot(n_ref[...], wa_ref[...], preferred_element_type=jnp.float32)
    acc += jnp.dot(s_ref[...], wb_ref[...], preferred_element_type=jnp.float32)
    acc += jnp.dot(r_ref[...], wc_ref[...], preferred_element_type=jnp.float32)
    acc += jnp.dot(gl_ref[...], wg_ref[...], preferred_element_type=jnp.float32)
    h = jnp.maximum(acc + b1_ref[...], 0.0)
    o_ref[...] = (jnp.dot(h, w2_ref[...], preferred_element_type=jnp.float32)
                  + b2_ref[...] + n_ref[...])


def _tc_node(nodes, agg_s, agg_r, wa, wb, wc, wg, gl, b1, w2, b2):
    n, dn = nodes.shape
    no = w2.shape[1]
    blk = 1000
    grid = n // blk
    full = lambda i: (0, 0)
    return pl.pallas_call(
        _node_body,
        grid=(grid,),
        in_specs=[
            pl.BlockSpec((blk, dn), lambda i: (i, 0)),
            pl.BlockSpec((blk, agg_s.shape[1]), lambda i: (i, 0)),
            pl.BlockSpec((blk, agg_r.shape[1]), lambda i: (i, 0)),
            pl.BlockSpec(wa.shape, full),
            pl.BlockSpec(wb.shape, full),
            pl.BlockSpec(wc.shape, full),
            pl.BlockSpec(wg.shape, full),
            pl.BlockSpec(gl.shape, full),
            pl.BlockSpec(b1.shape, full),
            pl.BlockSpec(w2.shape, full),
            pl.BlockSpec(b2.shape, full),
        ],
        out_specs=pl.BlockSpec((blk, no), lambda i: (i, 0)),
        out_shape=jax.ShapeDtypeStruct((n, no), jnp.float32),
    )(nodes, agg_s, agg_r, wa, wb, wc, wg, gl, b1, w2, b2)


def kernel(nodes, edges, globals_, senders, receivers,
           W_e1, b_e1, W_e2, b_e2, W_n1, b_n1, W_n2, b_n2):
    n, dn = nodes.shape
    de = edges.shape[1]
    dg = globals_.shape[1]
    gl = globals_.reshape(1, dg).astype(jnp.float32)

    w1e = W_e1[:de]
    w1sr = W_e1[de:de + 2 * dn]
    we = w1sr[0::2]
    wo = w1sr[1::2]
    w1g = W_e1[de + 2 * dn:]

    nodes_pk = lax.bitcast_convert_type(
        nodes.astype(jnp.bfloat16).reshape(n, dn // 2, 2), jnp.int32)
    g_pk = _sc_gather(nodes_pk, senders, receivers)
    new_edges = _tc_edge(edges, g_pk, w1e, we, wo, w1g, gl,
                         b_e1.reshape(1, -1), W_e2, b_e2.reshape(1, -1))

    agg_s, agg_r = _sc_segsum2(new_edges, senders, receivers, n)

    eo = new_edges.shape[1]
    wa = W_n1[:dn]
    wb = W_n1[dn:dn + eo]
    wc = W_n1[dn + eo:dn + 2 * eo]
    wg = W_n1[dn + 2 * eo:]

    out_nodes = _tc_node(nodes, agg_s, agg_r, wa, wb, wc, wg, gl,
                         b_n1.reshape(1, -1), W_n2, b_n2.reshape(1, -1))
    return (out_nodes, edges, globals_)


# speedup vs baseline: 2.4901x; 1.0366x over previous
import functools

import jax
import jax.numpy as jnp
from jax import lax
from jax.experimental import pallas as pl
from jax.experimental.pallas import tpu as pltpu
from jax.experimental.pallas import tpu_sc as plsc

NC = 2
NS = 16
NW = NC * NS

_mesh = lambda: plsc.VectorSubcoreMesh(core_axis_name="c", subcore_axis_name="s")


def _sc_gather(nodes, senders, receivers):
    n, d = nodes.shape
    dt = nodes.dtype
    e = senders.shape[0]
    per_w = e // NW
    ch = 40
    ns = 5
    n_grp = per_w // (ch * ns)

    @functools.partial(
        pl.kernel,
        mesh=_mesh(),
        out_type=jax.ShapeDtypeStruct((e, 2 * d), dt),
        scratch_types=(
            [pltpu.VMEM((per_w,), jnp.int32)] * 2
            + [pltpu.VMEM((ch, d), dt)] * (2 * ns)
            + [pltpu.SemaphoreType.DMA] * (2 * ns)
        ),
    )
    def k(nodes_hbm, s_hbm, r_hbm, g_hbm, sidx, ridx, *bufs_sems):
        sbufs = bufs_sems[0:ns]
        rbufs = bufs_sems[ns:2 * ns]
        gsems = bufs_sems[2 * ns:3 * ns]
        wsems = bufs_sems[3 * ns:4 * ns]
        wid = lax.axis_index("s") * NC + lax.axis_index("c")
        base = wid * per_w
        pltpu.sync_copy(s_hbm.at[pl.ds(base, per_w)], sidx)
        pltpu.sync_copy(r_hbm.at[pl.ds(base, per_w)], ridx)

        def group(g, _):
            g0 = g * (ch * ns)
            gh = []
            for s in range(ns):
                o = g0 + s * ch
                gh.append((
                    pltpu.async_copy(
                        nodes_hbm.at[sidx.at[pl.ds(o, ch)]], sbufs[s],
                        gsems[s]),
                    pltpu.async_copy(
                        nodes_hbm.at[ridx.at[pl.ds(o, ch)]], rbufs[s],
                        gsems[s]),
                ))
            wh = []
            for s in range(ns):
                o = base + g0 + s * ch
                gh[s][0].wait()
                gh[s][1].wait()
                wh.append((
                    pltpu.async_copy(
                        sbufs[s], g_hbm.at[pl.ds(o, ch), pl.ds(0, d)],
                        wsems[s]),
                    pltpu.async_copy(
                        rbufs[s], g_hbm.at[pl.ds(o, ch), pl.ds(d, d)],
                        wsems[s]),
                ))
            for s in range(ns):
                wh[s][0].wait()
                wh[s][1].wait()
            return _

        lax.fori_loop(0, n_grp, group, 0)

    return k(nodes, senders, receivers)


def _sc_segsum2(vals, idx_a, idx_b, n_seg):
    f = vals.shape[1]
    e = idx_a.shape[0]
    fb = f // NC
    per_t = e // NS
    chs = (104, 96)
    pair = sum(chs)
    n_grp = per_t // pair
    n_pad = ((n_seg + NS * 8 - 1) // (NS * 8)) * (NS * 8)
    rows_t = n_pad // NS
    zeros = jnp.zeros((rows_t, fb), jnp.float32)
    out_sd = jax.ShapeDtypeStruct((n_pad, f), jnp.float32)

    @functools.partial(
        pl.kernel,
        mesh=_mesh(),
        out_type=(out_sd, out_sd),
        scratch_types=[
            pltpu.VMEM((chs[0],), jnp.int32),
            pltpu.VMEM((chs[1],), jnp.int32),
            pltpu.VMEM((chs[0], fb), jnp.float32),
            pltpu.VMEM((chs[1], fb), jnp.float32),
            pltpu.VMEM_SHARED((n_pad, fb), jnp.float32),
            pltpu.SemaphoreType.DMA,
            pltpu.SemaphoreType.DMA,
        ],
    )
    def k(v_hbm, ia_hbm, ib_hbm, z_hbm, outa_hbm, outb_hbm,
          ib0, ib1, rb0, rb1, table, vsem0, vsem1):
        c = lax.axis_index("c")
        sid = lax.axis_index("s")
        r0 = sid * rows_t
        ibufs = (ib0, ib1)
        rbufs = (rb0, rb1)
        vsems = (vsem0, vsem1)
        base = sid * per_t

        for i_hbm, out_hbm in ((ia_hbm, outa_hbm), (ib_hbm, outb_hbm)):
            pltpu.sync_copy(z_hbm, table.at[pl.ds(r0, rows_t)])
            plsc.subcore_barrier()

            def group(g, _):
                g0 = base + g * pair
                vh = []
                for s in range(2):
                    e0 = g0 + s * chs[0]
                    pltpu.sync_copy(i_hbm.at[pl.ds(e0, chs[s])], ibufs[s])
                    vh.append(pltpu.async_copy(
                        v_hbm.at[pl.ds(e0, chs[s]), pl.ds(c * fb, fb)],
                        rbufs[s], vsems[s]))
                for s in range(2):
                    vh[s].wait()
                    pltpu.sync_copy(rbufs[s], table.at[ibufs[s]], add=True)
                return _

            lax.fori_loop(0, n_grp, group, 0)

            plsc.subcore_barrier()
            pltpu.sync_copy(table.at[pl.ds(r0, rows_t)],
                            out_hbm.at[pl.ds(r0, rows_t), pl.ds(c * fb, fb)])

    agg_a, agg_b = k(vals, idx_a, idx_b, zeros)
    return agg_a[:n_seg], agg_b[:n_seg]


def _edge_body(e_ref, gpk_ref, w1e_ref, we_ref, wo_ref, b1_ref,
               w2_ref, b2_ref, o_ref):
    w = gpk_ref[...]
    lo = lax.bitcast_convert_type(w << 16, jnp.float32).astype(jnp.bfloat16)
    hi = lax.bitcast_convert_type(w & jnp.int32(-65536),
                                  jnp.float32).astype(jnp.bfloat16)
    acc = jnp.dot(e_ref[...].astype(jnp.bfloat16), w1e_ref[...],
                  preferred_element_type=jnp.float32)
    acc += jnp.dot(lo, we_ref[...], preferred_element_type=jnp.float32)
    acc += jnp.dot(hi, wo_ref[...], preferred_element_type=jnp.float32)
    h = jnp.maximum(acc + b1_ref[...], 0.0)
    o_ref[...] = (jnp.dot(h.astype(jnp.bfloat16), w2_ref[...],
                          preferred_element_type=jnp.float32)
                  + b2_ref[...])


def _tc_edge(edges, g_pk, w1e, we, wo, b1, w2, b2):
    e, de = edges.shape
    dg = g_pk.shape[1]
    eo = w2.shape[1]
    blk = 640
    grid = e // blk
    full = lambda i: (0, 0)
    return pl.pallas_call(
        _edge_body,
        grid=(grid,),
        in_specs=[
            pl.BlockSpec((blk, de), lambda i: (i, 0)),
            pl.BlockSpec((blk, dg), lambda i: (i, 0)),
            pl.BlockSpec(w1e.shape, full),
            pl.BlockSpec(we.shape, full),
            pl.BlockSpec(wo.shape, full),
            pl.BlockSpec(b1.shape, full),
            pl.BlockSpec(w2.shape, full),
            pl.BlockSpec(b2.shape, full),
        ],
        out_specs=pl.BlockSpec((blk, eo), lambda i: (i, 0)),
        out_shape=jax.ShapeDtypeStruct((e, eo), jnp.float32),
    )(edges, g_pk, w1e, we, wo, b1, w2, b2)


def _node_body(n_ref, s_ref, r_ref, wa_ref, wb_ref, wc_ref, wg_ref, gl_ref,
               b1_ref, w2_ref, b2_ref, o_ref):
    acc = jnp.dot(n_ref[...], wa_ref[...], preferred_element_type=jnp.float32)
    acc += jnp.dot(s_ref[...], wb_ref[...], preferred_element_type=jnp.float32)
    acc += jnp.dot(r_ref[...], wc_ref[...], preferred_element_type=jnp.float32)
    acc += jnp.dot(gl_ref[...], wg_ref[...], preferred_element_type=jnp.float32)
    h = jnp.maximum(acc + b1_ref[...], 0.0)
    o_ref[...] = (jnp.dot(h, w2_ref[...], preferred_element_type=jnp.float32)
                  + b2_ref[...] + n_ref[...])


def _tc_node(nodes, agg_s, agg_r, wa, wb, wc, wg, gl, b1, w2, b2):
    n, dn = nodes.shape
    no = w2.shape[1]
    blk = 1000
    grid = n // blk
    full = lambda i: (0, 0)
    return pl.pallas_call(
        _node_body,
        grid=(grid,),
        in_specs=[
            pl.BlockSpec((blk, dn), lambda i: (i, 0)),
            pl.BlockSpec((blk, agg_s.shape[1]), lambda i: (i, 0)),
            pl.BlockSpec((blk, agg_r.shape[1]), lambda i: (i, 0)),
            pl.BlockSpec(wa.shape, full),
            pl.BlockSpec(wb.shape, full),
            pl.BlockSpec(wc.shape, full),
            pl.BlockSpec(wg.shape, full),
            pl.BlockSpec(gl.shape, full),
            pl.BlockSpec(b1.shape, full),
            pl.BlockSpec(w2.shape, full),
            pl.BlockSpec(b2.shape, full),
        ],
        out_specs=pl.BlockSpec((blk, no), lambda i: (i, 0)),
        out_shape=jax.ShapeDtypeStruct((n, no), jnp.float32),
    )(nodes, agg_s, agg_r, wa, wb, wc, wg, gl, b1, w2, b2)


def kernel(nodes, edges, globals_, senders, receivers,
           W_e1, b_e1, W_e2, b_e2, W_n1, b_n1, W_n2, b_n2):
    n, dn = nodes.shape
    de = edges.shape[1]
    dg = globals_.shape[1]
    gl = globals_.reshape(1, dg).astype(jnp.float32)

    w1e = W_e1[:de].astype(jnp.bfloat16)
    w1sr = W_e1[de:de + 2 * dn]
    we = w1sr[0::2].astype(jnp.bfloat16)
    wo = w1sr[1::2].astype(jnp.bfloat16)
    w1g = W_e1[de + 2 * dn:]
    b1e = (gl @ w1g + b_e1.reshape(1, -1)).astype(jnp.float32)

    nodes_pk = lax.bitcast_convert_type(
        nodes.astype(jnp.bfloat16).reshape(n, dn // 2, 2), jnp.int32)
    g_pk = _sc_gather(nodes_pk, senders, receivers)
    new_edges = _tc_edge(edges, g_pk, w1e, we, wo,
                         b1e, W_e2.astype(jnp.bfloat16), b_e2.reshape(1, -1))

    agg_s, agg_r = _sc_segsum2(new_edges, senders, receivers, n)

    eo = new_edges.shape[1]
    wa = W_n1[:dn]
    wb = W_n1[dn:dn + eo]
    wc = W_n1[dn + eo:dn + 2 * eo]
    wg = W_n1[dn + 2 * eo:]

    out_nodes = _tc_node(nodes, agg_s, agg_r, wa, wb, wc, wg, gl,
                         b_n1.reshape(1, -1), W_n2, b_n2.reshape(1, -1))
    return (out_nodes, edges, globals_)


# speedup vs baseline: 2.7057x; 1.0866x over previous
import functools

import jax
import jax.numpy as jnp
from jax import lax
from jax.experimental import pallas as pl
from jax.experimental.pallas import tpu as pltpu
from jax.experimental.pallas import tpu_sc as plsc

NC = 2
NS = 16
NW = NC * NS

_mesh = lambda: plsc.VectorSubcoreMesh(core_axis_name="c", subcore_axis_name="s")


def _sc_gather(nodes, senders, receivers):
    n, d = nodes.shape
    dt = nodes.dtype
    e = senders.shape[0]
    per_w = e // NW
    ch = 40
    ns = 5
    n_grp = per_w // (ch * ns)

    @functools.partial(
        pl.kernel,
        mesh=_mesh(),
        out_type=jax.ShapeDtypeStruct((e, 2 * d), dt),
        scratch_types=(
            [pltpu.VMEM((per_w,), jnp.int32)] * 2
            + [pltpu.VMEM((ch, d), dt)] * (2 * ns)
            + [pltpu.SemaphoreType.DMA] * (2 * ns)
        ),
    )
    def k(nodes_hbm, s_hbm, r_hbm, g_hbm, sidx, ridx, *bufs_sems):
        sbufs = bufs_sems[0:ns]
        rbufs = bufs_sems[ns:2 * ns]
        gsems = bufs_sems[2 * ns:3 * ns]
        wsems = bufs_sems[3 * ns:4 * ns]
        wid = lax.axis_index("s") * NC + lax.axis_index("c")
        base = wid * per_w
        pltpu.sync_copy(s_hbm.at[pl.ds(base, per_w)], sidx)
        pltpu.sync_copy(r_hbm.at[pl.ds(base, per_w)], ridx)

        def group(g, _):
            g0 = g * (ch * ns)
            gh = []
            for s in range(ns):
                o = g0 + s * ch
                gh.append((
                    pltpu.async_copy(
                        nodes_hbm.at[sidx.at[pl.ds(o, ch)]], sbufs[s],
                        gsems[s]),
                    pltpu.async_copy(
                        nodes_hbm.at[ridx.at[pl.ds(o, ch)]], rbufs[s],
                        gsems[s]),
                ))
            wh = []
            for s in range(ns):
                o = base + g0 + s * ch
                gh[s][0].wait()
                gh[s][1].wait()
                wh.append((
                    pltpu.async_copy(
                        sbufs[s], g_hbm.at[pl.ds(o, ch), pl.ds(0, d)],
                        wsems[s]),
                    pltpu.async_copy(
                        rbufs[s], g_hbm.at[pl.ds(o, ch), pl.ds(d, d)],
                        wsems[s]),
                ))
            for s in range(ns):
                wh[s][0].wait()
                wh[s][1].wait()
            return _

        lax.fori_loop(0, n_grp, group, 0)

    return k(nodes, senders, receivers)


def _sc_segsum2(vals, idx_a, idx_b, n_seg):
    f = vals.shape[1]
    e = idx_a.shape[0]
    fb = f // NC
    per_t = e // NS
    chs = (104, 96)
    pair = sum(chs)
    n_grp = per_t // pair
    n_pad = ((n_seg + NS * 8 - 1) // (NS * 8)) * (NS * 8)
    rows_t = n_pad // NS
    zeros = jnp.zeros((rows_t, fb), jnp.float32)
    out_sd = jax.ShapeDtypeStruct((n_pad, f), jnp.float32)

    @functools.partial(
        pl.kernel,
        mesh=_mesh(),
        out_type=(out_sd, out_sd),
        scratch_types=[
            pltpu.VMEM((chs[0],), jnp.int32),
            pltpu.VMEM((chs[1],), jnp.int32),
            pltpu.VMEM((chs[0], fb), jnp.float32),
            pltpu.VMEM((chs[1], fb), jnp.float32),
            pltpu.VMEM_SHARED((n_pad, fb), jnp.float32),
            pltpu.SemaphoreType.DMA,
            pltpu.SemaphoreType.DMA,
        ],
    )
    def k(v_hbm, ia_hbm, ib_hbm, z_hbm, outa_hbm, outb_hbm,
          ib0, ib1, rb0, rb1, table, vsem0, vsem1):
        c = lax.axis_index("c")
        sid = lax.axis_index("s")
        r0 = sid * rows_t
        ibufs = (ib0, ib1)
        rbufs = (rb0, rb1)
        vsems = (vsem0, vsem1)
        base = sid * per_t

        for i_hbm, out_hbm in ((ia_hbm, outa_hbm), (ib_hbm, outb_hbm)):
            pltpu.sync_copy(z_hbm, table.at[pl.ds(r0, rows_t)])
            plsc.subcore_barrier()

            def group(g, _):
                g0 = base + g * pair
                vh = []
                for s in range(2):
                    e0 = g0 + s * chs[0]
                    pltpu.sync_copy(i_hbm.at[pl.ds(e0, chs[s])], ibufs[s])
                    vh.append(pltpu.async_copy(
                        v_hbm.at[pl.ds(e0, chs[s]), pl.ds(c * fb, fb)],
                        rbufs[s], vsems[s]))
                for s in range(2):
                    vh[s].wait()
                    pltpu.sync_copy(rbufs[s], table.at[ibufs[s]], add=True)
                return _

            lax.fori_loop(0, n_grp, group, 0)

            plsc.subcore_barrier()
            pltpu.sync_copy(table.at[pl.ds(r0, rows_t)],
                            out_hbm.at[pl.ds(r0, rows_t), pl.ds(c * fb, fb)])

    agg_a, agg_b = k(vals, idx_a, idx_b, zeros)
    return agg_a[:n_seg], agg_b[:n_seg]


def _edge_body(e_ref, gpk_ref, w1e_ref, we_ref, wo_ref, b1_ref,
               w2_ref, b2_ref, o_ref):
    w = gpk_ref[...]
    lo = lax.bitcast_convert_type(w << 16, jnp.float32).astype(jnp.bfloat16)
    hi = lax.bitcast_convert_type(w & jnp.int32(-65536),
                                  jnp.float32).astype(jnp.bfloat16)
    acc = jnp.dot(e_ref[...].astype(jnp.bfloat16), w1e_ref[...],
                  preferred_element_type=jnp.float32)
    acc += jnp.dot(lo, we_ref[...], preferred_element_type=jnp.float32)
    acc += jnp.dot(hi, wo_ref[...], preferred_element_type=jnp.float32)
    h = jnp.maximum(acc + b1_ref[...], 0.0)
    o_ref[...] = (jnp.dot(h.astype(jnp.bfloat16), w2_ref[...],
                          preferred_element_type=jnp.float32)
                  + b2_ref[...])


def _tc_edge(edges, g_pk, w1e, we, wo, b1, w2, b2):
    e, de = edges.shape
    dg = g_pk.shape[1]
    eo = w2.shape[1]
    blk = 640
    grid = e // blk
    full = lambda i: (0, 0)
    return pl.pallas_call(
        _edge_body,
        grid=(grid,),
        in_specs=[
            pl.BlockSpec((blk, de), lambda i: (i, 0)),
            pl.BlockSpec((blk, dg), lambda i: (i, 0)),
            pl.BlockSpec(w1e.shape, full),
            pl.BlockSpec(we.shape, full),
            pl.BlockSpec(wo.shape, full),
            pl.BlockSpec(b1.shape, full),
            pl.BlockSpec(w2.shape, full),
            pl.BlockSpec(b2.shape, full),
        ],
        out_specs=pl.BlockSpec((blk, eo), lambda i: (i, 0)),
        out_shape=jax.ShapeDtypeStruct((e, eo), jnp.float32),
    )(edges, g_pk, w1e, we, wo, b1, w2, b2)


def _node_body(n_ref, sa_ref, sb_ref, ra_ref, rb_ref, wa_ref, wb_ref, wc_ref,
               b1_ref, w2_ref, b2_ref, o_ref):
    acc = jnp.dot(n_ref[...], wa_ref[...], preferred_element_type=jnp.float32)
    acc += jnp.dot(sa_ref[...] + sb_ref[...], wb_ref[...],
                   preferred_element_type=jnp.float32)
    acc += jnp.dot(ra_ref[...] + rb_ref[...], wc_ref[...],
                   preferred_element_type=jnp.float32)
    h = jnp.maximum(acc + b1_ref[...], 0.0)
    o_ref[...] = (jnp.dot(h, w2_ref[...], preferred_element_type=jnp.float32)
                  + b2_ref[...] + n_ref[...])


def _tc_node(nodes, aggs, wa, wb, wc, b1, w2, b2):
    n, dn = nodes.shape
    no = w2.shape[1]
    blk = 1000
    grid = n // blk
    full = lambda i: (0, 0)
    return pl.pallas_call(
        _node_body,
        grid=(grid,),
        in_specs=[
            pl.BlockSpec((blk, dn), lambda i: (i, 0)),
        ] + [
            pl.BlockSpec((blk, a.shape[1]), lambda i: (i, 0)) for a in aggs
        ] + [
            pl.BlockSpec(wa.shape, full),
            pl.BlockSpec(wb.shape, full),
            pl.BlockSpec(wc.shape, full),
            pl.BlockSpec(b1.shape, full),
            pl.BlockSpec(w2.shape, full),
            pl.BlockSpec(b2.shape, full),
        ],
        out_specs=pl.BlockSpec((blk, no), lambda i: (i, 0)),
        out_shape=jax.ShapeDtypeStruct((n, no), jnp.float32),
    )(nodes, *aggs, wa, wb, wc, b1, w2, b2)


def kernel(nodes, edges, globals_, senders, receivers,
           W_e1, b_e1, W_e2, b_e2, W_n1, b_n1, W_n2, b_n2):
    n, dn = nodes.shape
    e = senders.shape[0]
    de = edges.shape[1]
    dg = globals_.shape[1]
    gl = globals_.reshape(1, dg).astype(jnp.float32)

    w1e = W_e1[:de].astype(jnp.bfloat16)
    w1sr = W_e1[de:de + 2 * dn]
    we = w1sr[0::2].astype(jnp.bfloat16)
    wo = w1sr[1::2].astype(jnp.bfloat16)
    w1g = W_e1[de + 2 * dn:]
    b1e = (gl @ w1g + b_e1.reshape(1, -1)).astype(jnp.float32)

    nodes_pk = lax.bitcast_convert_type(
        nodes.astype(jnp.bfloat16).reshape(n, dn // 2, 2), jnp.int32)

    ea = 102400
    parts = []
    for lo_e, hi_e in ((0, ea), (ea, e)):
        s_p = senders[lo_e:hi_e]
        r_p = receivers[lo_e:hi_e]
        g_pk = _sc_gather(nodes_pk, s_p, r_p)
        ne_p = _tc_edge(edges[lo_e:hi_e], g_pk, w1e, we, wo,
                        b1e, W_e2.astype(jnp.bfloat16), b_e2.reshape(1, -1))
        parts.append(_sc_segsum2(ne_p, s_p, r_p, n))

    (agg_sa, agg_ra), (agg_sb, agg_rb) = parts

    eo = W_e2.shape[1]
    wa = W_n1[:dn]
    wb = W_n1[dn:dn + eo]
    wc = W_n1[dn + eo:dn + 2 * eo]
    wg = W_n1[dn + 2 * eo:]
    b1n = (gl @ wg + b_n1.reshape(1, -1)).astype(jnp.float32)

    out_nodes = _tc_node(nodes, (agg_sa, agg_sb, agg_ra, agg_rb),
                         wa, wb, wc, b1n, W_n2, b_n2.reshape(1, -1))
    return (out_nodes, edges, globals_)


# speedup vs baseline: 2.9199x; 1.0792x over previous
import functools

import jax
import jax.numpy as jnp
from jax import lax
from jax.experimental import pallas as pl
from jax.experimental.pallas import tpu as pltpu
from jax.experimental.pallas import tpu_sc as plsc

NC = 2
NS = 16
NW = NC * NS

_mesh = lambda: plsc.VectorSubcoreMesh(core_axis_name="c", subcore_axis_name="s")


def _sc_gather(nodes, senders, receivers):
    n, d = nodes.shape
    dt = nodes.dtype
    e = senders.shape[0]
    per_w = e // NW
    ch = 40
    ns = 5
    n_grp = per_w // (ch * ns)

    @functools.partial(
        pl.kernel,
        mesh=_mesh(),
        out_type=jax.ShapeDtypeStruct((e, 2 * d), dt),
        scratch_types=(
            [pltpu.VMEM((per_w,), jnp.int32)] * 2
            + [pltpu.VMEM((ch, d), dt)] * (2 * ns)
            + [pltpu.SemaphoreType.DMA] * (2 * ns)
        ),
    )
    def k(nodes_hbm, s_hbm, r_hbm, g_hbm, sidx, ridx, *bufs_sems):
        sbufs = bufs_sems[0:ns]
        rbufs = bufs_sems[ns:2 * ns]
        gsems = bufs_sems[2 * ns:3 * ns]
        wsems = bufs_sems[3 * ns:4 * ns]
        wid = lax.axis_index("s") * NC + lax.axis_index("c")
        base = wid * per_w
        pltpu.sync_copy(s_hbm.at[pl.ds(base, per_w)], sidx)
        pltpu.sync_copy(r_hbm.at[pl.ds(base, per_w)], ridx)

        def group(g, _):
            g0 = g * (ch * ns)
            gh = []
            for s in range(ns):
                o = g0 + s * ch
                gh.append((
                    pltpu.async_copy(
                        nodes_hbm.at[sidx.at[pl.ds(o, ch)]], sbufs[s],
                        gsems[s]),
                    pltpu.async_copy(
                        nodes_hbm.at[ridx.at[pl.ds(o, ch)]], rbufs[s],
                        gsems[s]),
                ))
            wh = []
            for s in range(ns):
                o = base + g0 + s * ch
                gh[s][0].wait()
                gh[s][1].wait()
                wh.append((
                    pltpu.async_copy(
                        sbufs[s], g_hbm.at[pl.ds(o, ch), pl.ds(0, d)],
                        wsems[s]),
                    pltpu.async_copy(
                        rbufs[s], g_hbm.at[pl.ds(o, ch), pl.ds(d, d)],
                        wsems[s]),
                ))
            for s in range(ns):
                wh[s][0].wait()
                wh[s][1].wait()
            return _

        lax.fori_loop(0, n_grp, group, 0)

    return k(nodes, senders, receivers)


def _sc_segsum2(vals, idx_a, idx_b, n_seg):
    f = vals.shape[1]
    e = idx_a.shape[0]
    fb = f // NC
    per_t = e // NS
    chs = (104, 96)
    pair = sum(chs)
    n_grp = per_t // pair
    n_pad = ((n_seg + NS * 8 - 1) // (NS * 8)) * (NS * 8)
    rows_t = n_pad // NS
    zeros = jnp.zeros((rows_t, fb), jnp.float32)
    out_sd = jax.ShapeDtypeStruct((n_pad, f), jnp.float32)

    @functools.partial(
        pl.kernel,
        mesh=_mesh(),
        out_type=(out_sd, out_sd),
        scratch_types=[
            pltpu.VMEM((chs[0],), jnp.int32),
            pltpu.VMEM((chs[1],), jnp.int32),
            pltpu.VMEM((chs[0], fb), jnp.float32),
            pltpu.VMEM((chs[1], fb), jnp.float32),
            pltpu.VMEM_SHARED((n_pad, fb), jnp.float32),
            pltpu.SemaphoreType.DMA,
            pltpu.SemaphoreType.DMA,
        ],
    )
    def k(v_hbm, ia_hbm, ib_hbm, z_hbm, outa_hbm, outb_hbm,
          ib0, ib1, rb0, rb1, table, vsem0, vsem1):
        c = lax.axis_index("c")
        sid = lax.axis_index("s")
        r0 = sid * rows_t
        ibufs = (ib0, ib1)
        rbufs = (rb0, rb1)
        vsems = (vsem0, vsem1)
        base = sid * per_t

        for i_hbm, out_hbm in ((ia_hbm, outa_hbm), (ib_hbm, outb_hbm)):
            pltpu.sync_copy(z_hbm, table.at[pl.ds(r0, rows_t)])
            plsc.subcore_barrier()

            def group(g, _):
                g0 = base + g * pair
                vh = []
                for s in range(2):
                    e0 = g0 + s * chs[0]
                    pltpu.sync_copy(i_hbm.at[pl.ds(e0, chs[s])], ibufs[s])
                    vh.append(pltpu.async_copy(
                        v_hbm.at[pl.ds(e0, chs[s]), pl.ds(c * fb, fb)],
                        rbufs[s], vsems[s]))
                for s in range(2):
                    vh[s].wait()
                    pltpu.sync_copy(rbufs[s], table.at[ibufs[s]], add=True)
                return _

            lax.fori_loop(0, n_grp, group, 0)

            plsc.subcore_barrier()
            pltpu.sync_copy(table.at[pl.ds(r0, rows_t)],
                            out_hbm.at[pl.ds(r0, rows_t), pl.ds(c * fb, fb)])

    agg_a, agg_b = k(vals, idx_a, idx_b, zeros)
    return agg_a[:n_seg], agg_b[:n_seg]


def _edge_body(e_ref, gpk_ref, w1e_ref, we_ref, wo_ref, b1_ref,
               w2_ref, b2_ref, o_ref):
    w = gpk_ref[...]
    lo = lax.bitcast_convert_type(w << 16, jnp.float32).astype(jnp.bfloat16)
    hi = lax.bitcast_convert_type(w & jnp.int32(-65536),
                                  jnp.float32).astype(jnp.bfloat16)
    acc = jnp.dot(e_ref[...].astype(jnp.bfloat16), w1e_ref[...],
                  preferred_element_type=jnp.float32)
    acc += jnp.dot(lo, we_ref[...], preferred_element_type=jnp.float32)
    acc += jnp.dot(hi, wo_ref[...], preferred_element_type=jnp.float32)
    h = jnp.maximum(acc + b1_ref[...], 0.0)
    o_ref[...] = (jnp.dot(h.astype(jnp.bfloat16), w2_ref[...],
                          preferred_element_type=jnp.float32)
                  + b2_ref[...])


def _tc_edge(edges, g_pk, w1e, we, wo, b1, w2, b2):
    e, de = edges.shape
    dg = g_pk.shape[1]
    eo = w2.shape[1]
    blk = 640
    grid = e // blk
    full = lambda i: (0, 0)
    return pl.pallas_call(
        _edge_body,
        grid=(grid,),
        in_specs=[
            pl.BlockSpec((blk, de), lambda i: (i, 0)),
            pl.BlockSpec((blk, dg), lambda i: (i, 0)),
            pl.BlockSpec(w1e.shape, full),
            pl.BlockSpec(we.shape, full),
            pl.BlockSpec(wo.shape, full),
            pl.BlockSpec(b1.shape, full),
            pl.BlockSpec(w2.shape, full),
            pl.BlockSpec(b2.shape, full),
        ],
        out_specs=pl.BlockSpec((blk, eo), lambda i: (i, 0)),
        out_shape=jax.ShapeDtypeStruct((e, eo), jnp.float32),
    )(edges, g_pk, w1e, we, wo, b1, w2, b2)


def _make_node_body(np_):
    def _node_body(*refs):
        n_ref = refs[0]
        s_refs = refs[1:1 + np_]
        r_refs = refs[1 + np_:1 + 2 * np_]
        wa_ref, wb_ref, wc_ref, b1_ref, w2_ref, b2_ref, o_ref = refs[1 + 2 * np_:]
        acc = jnp.dot(n_ref[...], wa_ref[...],
                      preferred_element_type=jnp.float32)
        s_sum = s_refs[0][...]
        r_sum = r_refs[0][...]
        for sr in s_refs[1:]:
            s_sum += sr[...]
        for rr in r_refs[1:]:
            r_sum += rr[...]
        acc += jnp.dot(s_sum, wb_ref[...], preferred_element_type=jnp.float32)
        acc += jnp.dot(r_sum, wc_ref[...], preferred_element_type=jnp.float32)
        h = jnp.maximum(acc + b1_ref[...], 0.0)
        o_ref[...] = (jnp.dot(h, w2_ref[...],
                              preferred_element_type=jnp.float32)
                      + b2_ref[...] + n_ref[...])
    return _node_body


def _tc_node(nodes, aggs, wa, wb, wc, b1, w2, b2):
    n, dn = nodes.shape
    no = w2.shape[1]
    blk = 1000
    grid = n // blk
    full = lambda i: (0, 0)
    return pl.pallas_call(
        _make_node_body(len(aggs) // 2),
        grid=(grid,),
        in_specs=[
            pl.BlockSpec((blk, dn), lambda i: (i, 0)),
        ] + [
            pl.BlockSpec((blk, a.shape[1]), lambda i: (i, 0)) for a in aggs
        ] + [
            pl.BlockSpec(wa.shape, full),
            pl.BlockSpec(wb.shape, full),
            pl.BlockSpec(wc.shape, full),
            pl.BlockSpec(b1.shape, full),
            pl.BlockSpec(w2.shape, full),
            pl.BlockSpec(b2.shape, full),
        ],
        out_specs=pl.BlockSpec((blk, no), lambda i: (i, 0)),
        out_shape=jax.ShapeDtypeStruct((n, no), jnp.float32),
    )(nodes, *aggs, wa, wb, wc, b1, w2, b2)


def kernel(nodes, edges, globals_, senders, receivers,
           W_e1, b_e1, W_e2, b_e2, W_n1, b_n1, W_n2, b_n2):
    n, dn = nodes.shape
    e = senders.shape[0]
    de = edges.shape[1]
    dg = globals_.shape[1]
    gl = globals_.reshape(1, dg).astype(jnp.float32)

    w1e = W_e1[:de].astype(jnp.bfloat16)
    w1sr = W_e1[de:de + 2 * dn]
    we = w1sr[0::2].astype(jnp.bfloat16)
    wo = w1sr[1::2].astype(jnp.bfloat16)
    w1g = W_e1[de + 2 * dn:]
    b1e = (gl @ w1g + b_e1.reshape(1, -1)).astype(jnp.float32)

    nodes_pk = lax.bitcast_convert_type(
        nodes.astype(jnp.bfloat16).reshape(n, dn // 2, 2), jnp.int32)

    cuts = (0, 57600, 108800, e)
    parts = []
    for lo_e, hi_e in zip(cuts[:-1], cuts[1:]):
        s_p = senders[lo_e:hi_e]
        r_p = receivers[lo_e:hi_e]
        g_pk = _sc_gather(nodes_pk, s_p, r_p)
        ne_p = _tc_edge(edges[lo_e:hi_e], g_pk, w1e, we, wo,
                        b1e, W_e2.astype(jnp.bfloat16), b_e2.reshape(1, -1))
        parts.append(_sc_segsum2(ne_p, s_p, r_p, n))

    s_parts = tuple(p[0] for p in parts)
    r_parts = tuple(p[1] for p in parts)

    eo = W_e2.shape[1]
    wa = W_n1[:dn]
    wb = W_n1[dn:dn + eo]
    wc = W_n1[dn + eo:dn + 2 * eo]
    wg = W_n1[dn + 2 * eo:]
    b1n = (gl @ wg + b_n1.reshape(1, -1)).astype(jnp.float32)

    out_nodes = _tc_node(nodes, s_parts + r_parts,
                         wa, wb, wc, b1n, W_n2, b_n2.reshape(1, -1))
    return (out_nodes, edges, globals_)
